# initial kernel scaffold (unmeasured)
import jax
import jax.numpy as jnp
from jax import lax
from jax.experimental import pallas as pl
from jax.experimental.pallas import tpu as pltpu


def kernel(
    x,
):
    def body(*refs):
        pass

    out_shape = jax.ShapeDtypeStruct(..., jnp.float32)
    return pl.pallas_call(body, out_shape=out_shape)(...)


# baseline (device time: 32515 ns/iter reference)
import jax
import jax.numpy as jnp
from jax import lax
from jax.experimental import pallas as pl
from jax.experimental.pallas import tpu as pltpu


def kernel(x):
    m, n = x.shape
    bf16 = jnp.bfloat16

    def body(x_ref, out_ref, xsend, xrecv, ysend, yrecv, sems):
        my_x = lax.axis_index("x")
        my_y = lax.axis_index("y")
        xnbr = (1 - my_x, my_y)
        ynbr = (my_x, 1 - my_y)

        barrier = pltpu.get_barrier_semaphore()
        for nbr in (xnbr, ynbr):
            pl.semaphore_signal(
                barrier, inc=1, device_id=nbr,
                device_id_type=pl.DeviceIdType.MESH,
            )
        pl.semaphore_wait(barrier, 2)

        xsend[...] = x_ref[...].astype(bf16)
        rdma_x = pltpu.make_async_remote_copy(
            src_ref=xsend, dst_ref=xrecv,
            send_sem=sems.at[0], recv_sem=sems.at[1],
            device_id=xnbr, device_id_type=pl.DeviceIdType.MESH,
        )
        rdma_x.start()
        rdma_x.wait()

        ysend[...] = xsend[...] + xrecv[...]
        rdma_y = pltpu.make_async_remote_copy(
            src_ref=ysend, dst_ref=yrecv,
            send_sem=sems.at[2], recv_sem=sems.at[3],
            device_id=ynbr, device_id_type=pl.DeviceIdType.MESH,
        )
        rdma_y.start()
        rdma_y.wait()

        @pl.when(my_y == 0)
        def _():
            out_ref[:, 0:n] = ysend[...].astype(jnp.float32)
            out_ref[:, n : 2 * n] = yrecv[...].astype(jnp.float32)

        @pl.when(my_y == 1)
        def _():
            out_ref[:, 0:n] = yrecv[...].astype(jnp.float32)
            out_ref[:, n : 2 * n] = ysend[...].astype(jnp.float32)

    return pl.pallas_call(
        body,
        out_shape=jax.ShapeDtypeStruct((m, 2 * n), jnp.float32),
        in_specs=[pl.BlockSpec(memory_space=pltpu.VMEM)],
        out_specs=pl.BlockSpec(memory_space=pltpu.VMEM),
        scratch_shapes=[
            pltpu.VMEM((m, n), bf16),
            pltpu.VMEM((m, n), bf16),
            pltpu.VMEM((m, n), bf16),
            pltpu.VMEM((m, n), bf16),
            pltpu.SemaphoreType.DMA((4,)),
        ],
        compiler_params=pltpu.CompilerParams(collective_id=0),
    )(x)


# device time: 22495 ns/iter; 1.4454x vs baseline; 1.4454x over previous
import jax
import jax.numpy as jnp
from jax import lax
from jax.experimental import pallas as pl
from jax.experimental.pallas import tpu as pltpu

C = 8


def kernel(x):
    m, n = x.shape
    mc = m // C
    bf16 = jnp.bfloat16

    def body(x_ref, out_ref, xsend, xrecv, ysend, yrecv,
             xs_send, xs_recv, ys_send, ys_recv):
        my_x = lax.axis_index("x")
        my_y = lax.axis_index("y")
        xnbr = (1 - my_x, my_y)
        ynbr = (my_x, 1 - my_y)

        barrier = pltpu.get_barrier_semaphore()
        for nbr in (xnbr, ynbr):
            pl.semaphore_signal(
                barrier, inc=1, device_id=nbr,
                device_id_type=pl.DeviceIdType.MESH,
            )
        pl.semaphore_wait(barrier, 2)

        def x_rdma(c):
            return pltpu.make_async_remote_copy(
                src_ref=xsend.at[c], dst_ref=xrecv.at[c],
                send_sem=xs_send.at[c], recv_sem=xs_recv.at[c],
                device_id=xnbr, device_id_type=pl.DeviceIdType.MESH,
            )

        def y_rdma(c):
            return pltpu.make_async_remote_copy(
                src_ref=ysend.at[c], dst_ref=yrecv.at[c],
                send_sem=ys_send.at[c], recv_sem=ys_recv.at[c],
                device_id=ynbr, device_id_type=pl.DeviceIdType.MESH,
            )

        for c in range(C):
            xsend[c, :, :] = x_ref[pl.ds(c * mc, mc), :].astype(bf16)
            x_rdma(c).start()

        for c in range(C):
            x_rdma(c).wait_recv()
            ysend[c, :, :] = xsend[c, :, :] + xrecv[c, :, :]
            y_rdma(c).start()

        for c in range(C):

            @pl.when(my_y == 0)
            def _():
                out_ref[pl.ds(c * mc, mc), 0:n] = ysend[c].astype(jnp.float32)

            @pl.when(my_y == 1)
            def _():
                out_ref[pl.ds(c * mc, mc), n : 2 * n] = ysend[c].astype(
                    jnp.float32
                )

        for c in range(C):
            y_rdma(c).wait_recv()

            @pl.when(my_y == 0)
            def _():
                out_ref[pl.ds(c * mc, mc), n : 2 * n] = yrecv[c].astype(
                    jnp.float32
                )

            @pl.when(my_y == 1)
            def _():
                out_ref[pl.ds(c * mc, mc), 0:n] = yrecv[c].astype(jnp.float32)

        for c in range(C):
            x_rdma(c).wait_send()
            y_rdma(c).wait_send()

    return pl.pallas_call(
        body,
        out_shape=jax.ShapeDtypeStruct((m, 2 * n), jnp.float32),
        in_specs=[pl.BlockSpec(memory_space=pltpu.VMEM)],
        out_specs=pl.BlockSpec(memory_space=pltpu.VMEM),
        scratch_shapes=[
            pltpu.VMEM((C, mc, n), bf16),
            pltpu.VMEM((C, mc, n), bf16),
            pltpu.VMEM((C, mc, n), bf16),
            pltpu.VMEM((C, mc, n), bf16),
            pltpu.SemaphoreType.DMA((C,)),
            pltpu.SemaphoreType.DMA((C,)),
            pltpu.SemaphoreType.DMA((C,)),
            pltpu.SemaphoreType.DMA((C,)),
        ],
        compiler_params=pltpu.CompilerParams(collective_id=0),
    )(x)


# device time: 21864 ns/iter; 1.4871x vs baseline; 1.0289x over previous
import jax
import jax.numpy as jnp
from jax import lax
from jax.experimental import pallas as pl
from jax.experimental.pallas import tpu as pltpu

C = 8


def kernel(x):
    m, n = x.shape
    mc = m // C
    bf16 = jnp.bfloat16

    def body(x_ref, out_ref, xsend, xrecv, xs_send, xs_recv, ys_send, ys_recv):
        my_x = lax.axis_index("x")
        my_y = lax.axis_index("y")
        xnbr = (1 - my_x, my_y)
        ynbr = (my_x, 1 - my_y)

        xsend[0, :, :] = x_ref[pl.ds(0, mc), :].astype(bf16)

        barrier = pltpu.get_barrier_semaphore()
        for nbr in (xnbr, ynbr):
            pl.semaphore_signal(
                barrier, inc=1, device_id=nbr,
                device_id_type=pl.DeviceIdType.MESH,
            )
        pl.semaphore_wait(barrier, 2)

        def x_rdma(c):
            return pltpu.make_async_remote_copy(
                src_ref=xsend.at[c], dst_ref=xrecv.at[c],
                send_sem=xs_send.at[c], recv_sem=xs_recv.at[c],
                device_id=xnbr, device_id_type=pl.DeviceIdType.MESH,
            )

        def y_rdma(c, off):
            sl = out_ref.at[pl.ds(c * mc, mc), pl.ds(off, n)]
            return pltpu.make_async_remote_copy(
                src_ref=sl, dst_ref=sl,
                send_sem=ys_send.at[c], recv_sem=ys_recv.at[c],
                device_id=ynbr, device_id_type=pl.DeviceIdType.MESH,
            )

        for c in range(C):
            if c > 0:
                xsend[c, :, :] = x_ref[pl.ds(c * mc, mc), :].astype(bf16)
            x_rdma(c).start()

        for off_val, cond in ((0, my_y == 0), (n, my_y == 1)):

            @pl.when(cond)
            def _(off=off_val):
                for c in range(C):
                    x_rdma(c).wait_recv()
                    out_ref[pl.ds(c * mc, mc), pl.ds(off, n)] = (
                        xsend[c, :, :] + xrecv[c, :, :]
                    )
                    y_rdma(c, off).start()
                for c in range(C):
                    y_rdma(c, off).wait_recv()
                    y_rdma(c, off).wait_send()

        for c in range(C):
            x_rdma(c).wait_send()

    return pl.pallas_call(
        body,
        out_shape=jax.ShapeDtypeStruct((m, 2 * n), bf16),
        in_specs=[pl.BlockSpec(memory_space=pltpu.VMEM)],
        out_specs=pl.BlockSpec(memory_space=pltpu.VMEM),
        scratch_shapes=[
            pltpu.VMEM((C, mc, n), bf16),
            pltpu.VMEM((C, mc, n), bf16),
            pltpu.SemaphoreType.DMA((C,)),
            pltpu.SemaphoreType.DMA((C,)),
            pltpu.SemaphoreType.DMA((C,)),
            pltpu.SemaphoreType.DMA((C,)),
        ],
        compiler_params=pltpu.CompilerParams(collective_id=0),
    )(x)


# device time: 21427 ns/iter; 1.5175x vs baseline; 1.0204x over previous
import jax
import jax.numpy as jnp
from jax import lax
from jax.experimental import pallas as pl
from jax.experimental.pallas import tpu as pltpu

C = 16


def kernel(x):
    m, n = x.shape
    mc = m // C
    bf16 = jnp.bfloat16

    def body(x_ref, out_ref, xsend, xrecv, xs_send, xs_recv, ys_send, ys_recv):
        my_x = lax.axis_index("x")
        my_y = lax.axis_index("y")
        xnbr = (1 - my_x, my_y)
        ynbr = (my_x, 1 - my_y)

        xsend[0, :, :] = x_ref[pl.ds(0, mc), :].astype(bf16)

        barrier = pltpu.get_barrier_semaphore()
        for nbr in (xnbr, ynbr):
            pl.semaphore_signal(
                barrier, inc=1, device_id=nbr,
                device_id_type=pl.DeviceIdType.MESH,
            )
        pl.semaphore_wait(barrier, 2)

        def x_rdma(c):
            return pltpu.make_async_remote_copy(
                src_ref=xsend.at[c], dst_ref=xrecv.at[c],
                send_sem=xs_send.at[c], recv_sem=xs_recv.at[c],
                device_id=xnbr, device_id_type=pl.DeviceIdType.MESH,
            )

        def y_rdma(c, off):
            sl = out_ref.at[pl.ds(c * mc, mc), pl.ds(off, n)]
            return pltpu.make_async_remote_copy(
                src_ref=sl, dst_ref=sl,
                send_sem=ys_send.at[c], recv_sem=ys_recv.at[c],
                device_id=ynbr, device_id_type=pl.DeviceIdType.MESH,
            )

        for c in range(C):
            if c > 0:
                xsend[c, :, :] = x_ref[pl.ds(c * mc, mc), :].astype(bf16)
            x_rdma(c).start()

        for off_val, cond in ((0, my_y == 0), (n, my_y == 1)):

            @pl.when(cond)
            def _(off=off_val):
                for c in range(C):
                    x_rdma(c).wait_recv()
                    out_ref[pl.ds(c * mc, mc), pl.ds(off, n)] = (
                        xsend[c, :, :] + xrecv[c, :, :]
                    )
                    y_rdma(c, off).start()
                for c in range(C):
                    y_rdma(c, off).wait_recv()
                    y_rdma(c, off).wait_send()

        for c in range(C):
            x_rdma(c).wait_send()

    return pl.pallas_call(
        body,
        out_shape=jax.ShapeDtypeStruct((m, 2 * n), bf16),
        in_specs=[pl.BlockSpec(memory_space=pltpu.VMEM)],
        out_specs=pl.BlockSpec(memory_space=pltpu.VMEM),
        scratch_shapes=[
            pltpu.VMEM((C, mc, n), bf16),
            pltpu.VMEM((C, mc, n), bf16),
            pltpu.SemaphoreType.DMA((C,)),
            pltpu.SemaphoreType.DMA((C,)),
            pltpu.SemaphoreType.DMA((C,)),
            pltpu.SemaphoreType.DMA((C,)),
        ],
        compiler_params=pltpu.CompilerParams(collective_id=0),
    )(x)
